# Initial kernel scaffold; baseline (speedup 1.0000x reference)
import functools

import jax
import jax.numpy as jnp
from jax import lax
from jax.experimental import pallas as pl
from jax.experimental.pallas import tpu as pltpu
from jax.experimental.pallas import tpu_sc as plsc

NC, NS = 2, 16
NW = NC * NS
IPS = 128
G = 8
C = G * IPS


@functools.partial(jax.jit, static_argnums=(2, 3))
def _sc_gather(idx2d, table, B, D):
    b_per_w = B // NW
    n_chunks = b_per_w // C
    mesh = plsc.VectorSubcoreMesh(core_axis_name="c", subcore_axis_name="s")

    @functools.partial(
        pl.kernel,
        mesh=mesh,
        out_type=jax.ShapeDtypeStruct((B, D), jnp.float32),
        scratch_types=[
            pltpu.VMEM((G, IPS), jnp.int32),
            pltpu.VMEM((C, D), jnp.float32),
            pltpu.SemaphoreType.DMA,
        ],
    )
    def k(idx_hbm, table_hbm, out_hbm, idx_v, rows_v, sem):
        wid = lax.axis_index("s") * NC + lax.axis_index("c")
        base_row = wid * b_per_w
        base_ir = wid * (b_per_w // IPS)

        @pl.loop(0, n_chunks)
        def _(i):
            pltpu.sync_copy(idx_hbm.at[pl.ds(base_ir + i * G, G)], idx_v)
            copies = [
                pltpu.async_copy(
                    table_hbm.at[idx_v.at[j]],
                    rows_v.at[pl.ds(j * IPS, IPS)],
                    sem,
                )
                for j in range(G)
            ]
            for cp in copies:
                cp.wait()
            pltpu.sync_copy(rows_v, out_hbm.at[pl.ds(base_row + i * C, C)])

    return k(idx2d, table)


def kernel(x, weight):
    B = x.size
    D = weight.shape[1]
    idx2d = x.reshape(B // IPS, IPS).astype(jnp.int32)
    out = _sc_gather(idx2d, weight, B, D)
    return out.reshape(x.shape + (D,))


# speedup vs baseline: 4.8086x; 4.8086x over previous
import functools

import jax
import jax.numpy as jnp
from jax import lax
from jax.experimental import pallas as pl
from jax.experimental.pallas import tpu as pltpu
from jax.experimental.pallas import tpu_sc as plsc

NC, NS = 2, 16
NW = NC * NS
IPS = 128
G = 8
C = G * IPS


@functools.partial(jax.jit, static_argnums=(2, 3))
def _sc_gather(idx2d, table, B, D):
    b_per_w = B // NW
    n_chunks = b_per_w // C
    mesh = plsc.VectorSubcoreMesh(core_axis_name="c", subcore_axis_name="s")

    @functools.partial(
        pl.kernel,
        mesh=mesh,
        out_type=jax.ShapeDtypeStruct((B, D), jnp.float32),
        scratch_types=[
            pltpu.VMEM((G, IPS), jnp.int32),
            pltpu.VMEM((C, D), jnp.float32),
            pltpu.SemaphoreType.DMA,
        ],
        compiler_params=pltpu.CompilerParams(use_tc_tiling_on_sc=False),
    )
    def k(idx_hbm, table_hbm, out_hbm, idx_v, rows_v, sem):
        wid = lax.axis_index("s") * NC + lax.axis_index("c")
        base_row = wid * b_per_w
        base_ir = wid * (b_per_w // IPS)

        @pl.loop(0, n_chunks)
        def _(i):
            pltpu.sync_copy(idx_hbm.at[pl.ds(base_ir + i * G, G)], idx_v)
            copies = [
                pltpu.async_copy(
                    table_hbm.at[idx_v.at[j]],
                    rows_v.at[pl.ds(j * IPS, IPS)],
                    sem,
                )
                for j in range(G)
            ]
            for cp in copies:
                cp.wait()
            pltpu.sync_copy(rows_v, out_hbm.at[pl.ds(base_row + i * C, C)])

    return k(idx2d, table)


def kernel(x, weight):
    B = x.size
    D = weight.shape[1]
    idx2d = x.reshape(B // IPS, IPS).astype(jnp.int32)
    out = _sc_gather(idx2d, weight, B, D)
    return out.reshape(x.shape + (D,))


# speedup vs baseline: 5.0403x; 1.0482x over previous
import functools

import jax
import jax.numpy as jnp
from jax import lax
from jax.experimental import pallas as pl
from jax.experimental.pallas import tpu as pltpu
from jax.experimental.pallas import tpu_sc as plsc

NC, NS = 2, 16
NW = NC * NS
IPS = 128
G = 10
C = G * IPS
NBUF = 2


@functools.partial(jax.jit, static_argnums=(2, 3))
def _sc_gather(idx2d, table, B, D):
    b_per_w = B // NW
    n_chunks = b_per_w // C
    assert n_chunks % NBUF == 0
    mesh = plsc.VectorSubcoreMesh(core_axis_name="c", subcore_axis_name="s")

    @functools.partial(
        pl.kernel,
        mesh=mesh,
        out_type=jax.ShapeDtypeStruct((B, D), jnp.float32),
        scratch_types=[
            pltpu.VMEM((NBUF, G, IPS), jnp.int32),
            pltpu.VMEM((NBUF, C, D), jnp.float32),
            [pltpu.SemaphoreType.DMA] * NBUF,
            pltpu.SemaphoreType.DMA,
            [pltpu.SemaphoreType.DMA] * NBUF,
        ],
        compiler_params=pltpu.CompilerParams(use_tc_tiling_on_sc=False),
    )
    def k(idx_hbm, table_hbm, out_hbm, idx_v, rows_v, i_sems, g_sem, s_sems):
        wid = lax.axis_index("s") * NC + lax.axis_index("c")
        base_row = wid * b_per_w
        base_ir = wid * (b_per_w // IPS)

        def idx_copy(i, b):
            return pltpu.make_async_copy(
                idx_hbm.at[pl.ds(base_ir + i * G, G)], idx_v.at[b], i_sems[b])

        def store_copy(i, b):
            return pltpu.make_async_copy(
                rows_v.at[b], out_hbm.at[pl.ds(base_row + i * C, C)], s_sems[b])

        for b in range(NBUF):
            idx_copy(b, b).start()

        @pl.loop(0, n_chunks, step=NBUF)
        def _(i0):
            for b in range(NBUF):
                i = i0 + b
                @pl.when(i >= NBUF)
                def _():
                    store_copy(0, b).wait()
                idx_copy(0, b).wait()
                gathers = [
                    pltpu.async_copy(
                        table_hbm.at[idx_v.at[b, j]],
                        rows_v.at[b, pl.ds(j * IPS, IPS)],
                        g_sem,
                    )
                    for j in range(G)
                ]
                for cp in gathers:
                    cp.wait()
                @pl.when(i + NBUF < n_chunks)
                def _():
                    idx_copy(i + NBUF, b).start()
                store_copy(i, b).start()

        for b in range(NBUF):
            store_copy(0, b).wait()

    return k(idx2d, table)


def kernel(x, weight):
    B = x.size
    D = weight.shape[1]
    idx2d = x.reshape(B // IPS, IPS).astype(jnp.int32)
    out = _sc_gather(idx2d, weight, B, D)
    return out.reshape(x.shape + (D,))


# speedup vs baseline: 5.0518x; 1.0023x over previous
import functools

import jax
import jax.numpy as jnp
from jax import lax
from jax.experimental import pallas as pl
from jax.experimental.pallas import tpu as pltpu
from jax.experimental.pallas import tpu_sc as plsc

NC, NS = 2, 16
NW = NC * NS
IPS = 128
G = 10
C = G * IPS
NBUF = 2


@functools.partial(jax.jit, static_argnums=(2, 3))
def _sc_gather(idx2d, table, B, D):
    b_per_w = B // NW
    n_chunks = b_per_w // C
    assert n_chunks % NBUF == 0
    mesh = plsc.VectorSubcoreMesh(core_axis_name="c", subcore_axis_name="s")

    @functools.partial(
        pl.kernel,
        mesh=mesh,
        out_type=jax.ShapeDtypeStruct((B, D), jnp.float32),
        scratch_types=[
            pltpu.VMEM((NBUF, G, IPS), jnp.int32),
            pltpu.VMEM((NBUF, C, D), jnp.float32),
            [pltpu.SemaphoreType.DMA] * NBUF,
            [pltpu.SemaphoreType.DMA] * NBUF,
            [pltpu.SemaphoreType.DMA] * NBUF,
        ],
        compiler_params=pltpu.CompilerParams(use_tc_tiling_on_sc=False),
    )
    def k(idx_hbm, table_hbm, out_hbm, idx_v, rows_v, i_sems, g_sems, s_sems):
        wid = lax.axis_index("s") * NC + lax.axis_index("c")
        base_row = wid * b_per_w
        base_ir = wid * (b_per_w // IPS)

        def idx_copy(i, b):
            return pltpu.make_async_copy(
                idx_hbm.at[pl.ds(base_ir + i * G, G)], idx_v.at[b], i_sems[b])

        def store_copy(i, b):
            return pltpu.make_async_copy(
                rows_v.at[b], out_hbm.at[pl.ds(base_row + i * C, C)], s_sems[b])

        def fire_gathers(b):
            for j in range(G):
                pltpu.make_async_copy(
                    table_hbm.at[idx_v.at[b, j]],
                    rows_v.at[b, pl.ds(j * IPS, IPS)],
                    g_sems[b],
                ).start()

        def drain_gathers(b):
            for j in range(G):
                pltpu.make_async_copy(
                    table_hbm.at[idx_v.at[b, j]],
                    rows_v.at[b, pl.ds(j * IPS, IPS)],
                    g_sems[b],
                ).wait()

        idx_copy(0, 0).start()

        @pl.loop(0, n_chunks, step=NBUF)
        def _(i0):
            for b in range(NBUF):
                i = i0 + b
                o = 1 - b
                idx_copy(0, b).wait()
                @pl.when(i >= NBUF)
                def _():
                    store_copy(0, b).wait()
                fire_gathers(b)
                @pl.when(i >= 1)
                def _():
                    drain_gathers(o)
                    store_copy(i - 1, o).start()
                @pl.when(i + 1 < n_chunks)
                def _():
                    idx_copy(i + 1, o).start()

        last = (n_chunks - 1) % NBUF
        drain_gathers(last)
        store_copy(n_chunks - 1, last).start()
        for b in range(NBUF):
            store_copy(0, b).wait()

    return k(idx2d, table)


def kernel(x, weight):
    B = x.size
    D = weight.shape[1]
    idx2d = x.reshape(B // IPS, IPS).astype(jnp.int32)
    out = _sc_gather(idx2d, weight, B, D)
    return out.reshape(x.shape + (D,))


# speedup vs baseline: 5.0520x; 1.0000x over previous
import functools

import jax
import jax.numpy as jnp
from jax import lax
from jax.experimental import pallas as pl
from jax.experimental.pallas import tpu as pltpu
from jax.experimental.pallas import tpu_sc as plsc

NC, NS = 2, 16
NW = NC * NS
RB = 4
IPS = 40
NBUF = 2


@functools.partial(jax.jit, static_argnums=(2, 3, 4))
def _sc_gather(x, table, N1, N2, D):
    rows_per_w = N1 // NW
    n_chunks = rows_per_w // RB
    assert n_chunks % NBUF == 0 and N2 % IPS == 0
    mesh = plsc.VectorSubcoreMesh(core_axis_name="c", subcore_axis_name="s")

    @functools.partial(
        pl.kernel,
        mesh=mesh,
        out_type=jax.ShapeDtypeStruct((N1, N2, D), jnp.float32),
        scratch_types=[
            pltpu.VMEM((NBUF, RB, N2), jnp.int32),
            pltpu.VMEM((NBUF, RB, N2, D), jnp.float32),
            [pltpu.SemaphoreType.DMA] * NBUF,
            [pltpu.SemaphoreType.DMA] * NBUF,
            [pltpu.SemaphoreType.DMA] * NBUF,
        ],
        compiler_params=pltpu.CompilerParams(use_tc_tiling_on_sc=False),
    )
    def k(x_hbm, table_hbm, out_hbm, idx_v, rows_v, i_sems, g_sems, s_sems):
        wid = lax.axis_index("s") * NC + lax.axis_index("c")
        base_row = wid * rows_per_w

        def idx_copy(i, b):
            return pltpu.make_async_copy(
                x_hbm.at[pl.ds(base_row + i * RB, RB)], idx_v.at[b], i_sems[b])

        def store_copy(i, b):
            return pltpu.make_async_copy(
                rows_v.at[b], out_hbm.at[pl.ds(base_row + i * RB, RB)],
                s_sems[b])

        def gather_descs(b):
            return [
                pltpu.make_async_copy(
                    table_hbm.at[idx_v.at[b, r, pl.ds(h * IPS, IPS)]],
                    rows_v.at[b, r, pl.ds(h * IPS, IPS)],
                    g_sems[b],
                )
                for r in range(RB)
                for h in range(N2 // IPS)
            ]

        idx_copy(0, 0).start()

        @pl.loop(0, n_chunks, step=NBUF)
        def _(i0):
            for b in range(NBUF):
                i = i0 + b
                o = 1 - b
                idx_copy(0, b).wait()
                @pl.when(i >= NBUF)
                def _():
                    store_copy(0, b).wait()
                for d in gather_descs(b):
                    d.start()
                @pl.when(i >= 1)
                def _():
                    for d in gather_descs(o):
                        d.wait()
                    store_copy(i - 1, o).start()
                @pl.when(i + 1 < n_chunks)
                def _():
                    idx_copy(i + 1, o).start()

        last = (n_chunks - 1) % NBUF
        for d in gather_descs(last):
            d.wait()
        store_copy(n_chunks - 1, last).start()
        for b in range(NBUF):
            store_copy(0, b).wait()

    return k(x, table)


def kernel(x, weight):
    N1, N2 = x.shape
    D = weight.shape[1]
    return _sc_gather(x.astype(jnp.int32), weight, N1, N2, D)
